# Initial kernel scaffold; baseline (speedup 1.0000x reference)
import jax
import jax.numpy as jnp
from jax.experimental import pallas as pl


def kernel(x1, x2, table, W1, b1, W2, b2, Wfx, bfx):
    raise NotImplementedError("write your pallas kernel here")


# speedup vs baseline: 1.0956x; 1.0956x over previous
import functools
import jax
import jax.numpy as jnp
from jax import lax
from jax.experimental import pallas as pl
from jax.experimental.pallas import tpu as pltpu
from jax.experimental.pallas import tpu_sc as plsc

EMB = 16
NUM_F = 26
NC, NS = 2, 16
NW = NC * NS
IDX_PER_GATHER = 128


def _make_gather(total_rows):
    n_chunks = total_rows // IDX_PER_GATHER
    cpw = n_chunks // NW
    mesh = plsc.VectorSubcoreMesh(core_axis_name="c", subcore_axis_name="s")

    @functools.partial(
        pl.kernel,
        mesh=mesh,
        out_type=jax.ShapeDtypeStruct((total_rows, EMB), jnp.float32),
        scratch_types=[
            pltpu.VMEM((cpw, IDX_PER_GATHER), jnp.int32),
            pltpu.VMEM((IDX_PER_GATHER, EMB), jnp.float32),
            pltpu.SemaphoreType.DMA,
        ],
        compiler_params=pltpu.CompilerParams(use_tc_tiling_on_sc=False),
    )
    def gather_k(idx_hbm, table_hbm, out_hbm, idx_v, rows_v, sem):
        wid = lax.axis_index("s") * NC + lax.axis_index("c")
        row0 = wid * cpw
        pltpu.sync_copy(idx_hbm.at[pl.ds(row0, cpw)], idx_v)

        def body(j, carry):
            pltpu.async_copy(table_hbm.at[idx_v.at[j]], rows_v, sem).wait()
            pltpu.sync_copy(
                rows_v, out_hbm.at[pl.ds((row0 + j) * IDX_PER_GATHER,
                                         IDX_PER_GATHER)])
            return carry

        lax.fori_loop(0, cpw, body, 0)

    return gather_k


def _mlp_body(flat_ref, x2_ref, W1_ref, b1_ref, W2_ref, b2_ref, Wh_ref,
              Wx_ref, bfx_ref, S_ref, out_ref):
    flat = flat_ref[...]
    h = jnp.dot(flat, W1_ref[...], preferred_element_type=jnp.float32)
    h = jnp.maximum(h + b1_ref[...], 0.0)
    h = jnp.dot(h, W2_ref[...], preferred_element_type=jnp.float32)
    h = jnp.maximum(h + b2_ref[...], 0.0)
    z = (jnp.dot(h, Wh_ref[...], preferred_element_type=jnp.float32)
         + jnp.dot(x2_ref[...], Wx_ref[...], preferred_element_type=jnp.float32)
         + bfx_ref[...])
    z = jnp.maximum(z, 0.0)
    s = jnp.dot(flat, S_ref[...], preferred_element_type=jnp.float32)
    fm = (jnp.sum(s * s, axis=1, keepdims=True)
          - jnp.sum(flat * flat, axis=1, keepdims=True))
    out_ref[...] = jax.nn.sigmoid(z + 0.5 * fm)


def _mlp_call(flat, x2, W1, b1, W2, b2, Wh, Wx, bfx, S, block_b=512):
    b, in_dim = flat.shape
    grid = (b // block_b,)
    full = lambda shape: pl.BlockSpec(shape, lambda i: (0, 0))
    return pl.pallas_call(
        _mlp_body,
        grid=grid,
        in_specs=[
            pl.BlockSpec((block_b, in_dim), lambda i: (i, 0)),
            pl.BlockSpec((block_b, x2.shape[1]), lambda i: (i, 0)),
            full(W1.shape), full(b1.shape), full(W2.shape), full(b2.shape),
            full(Wh.shape), full(Wx.shape), full(bfx.shape), full(S.shape),
        ],
        out_specs=pl.BlockSpec((block_b, 1), lambda i: (i, 0)),
        out_shape=jax.ShapeDtypeStruct((b, 1), jnp.float32),
    )(flat, x2, W1, b1, W2, b2, Wh, Wx, bfx, S)


def kernel(x1, x2, table, W1, b1, W2, b2, Wfx, bfx):
    b = x1.shape[0]
    total_rows = b * NUM_F
    idx = x1.reshape(total_rows // IDX_PER_GATHER, IDX_PER_GATHER)
    idx = idx.astype(jnp.int32)

    rows = _make_gather(total_rows)(idx, table)
    flat = rows.reshape(b, NUM_F * EMB)

    S = jnp.tile(jnp.eye(EMB, dtype=jnp.float32), (NUM_F, 1))
    out = _mlp_call(
        flat, x2, W1, b1.reshape(1, -1), W2, b2.reshape(1, -1),
        Wfx[:W2.shape[1]], Wfx[W2.shape[1]:], bfx.reshape(1, 1), S)
    return out.reshape(-1)


# speedup vs baseline: 1.1800x; 1.0770x over previous
import functools
import jax
import jax.numpy as jnp
from jax import lax
from jax.experimental import pallas as pl
from jax.experimental.pallas import tpu as pltpu
from jax.experimental.pallas import tpu_sc as plsc

EMB = 16
NUM_F = 26
NC, NS = 2, 16
NW = NC * NS
IDX_PER_GATHER = 128


def _make_gather(batch):
    spw = batch // NW
    SPG = 16
    npg = spw // SPG
    mesh = plsc.VectorSubcoreMesh(core_axis_name="c", subcore_axis_name="s")

    @functools.partial(
        pl.kernel,
        mesh=mesh,
        out_type=jax.ShapeDtypeStruct((batch * NUM_F, EMB), jnp.float32),
        scratch_types=[
            pltpu.VMEM((spw, NUM_F), jnp.int32),
            pltpu.VMEM((SPG * NUM_F, EMB), jnp.float32),
            pltpu.SemaphoreType.DMA,
        ],
        compiler_params=pltpu.CompilerParams(use_tc_tiling_on_sc=False),
    )
    def gather_k(x1_hbm, table_hbm, out_hbm, idx_v, rows_v, sem):
        wid = lax.axis_index("s") * NC + lax.axis_index("c")
        s0 = wid * spw
        pltpu.sync_copy(x1_hbm.at[pl.ds(s0, spw)], idx_v)

        def body(g, carry):
            copies = []
            for t in range(SPG):
                copies.append(pltpu.async_copy(
                    table_hbm.at[idx_v.at[g * SPG + t]],
                    rows_v.at[pl.ds(t * NUM_F, NUM_F)], sem))
            for c in copies:
                c.wait()
            pltpu.sync_copy(
                rows_v,
                out_hbm.at[pl.ds((s0 + g * SPG) * NUM_F, SPG * NUM_F)])
            return carry

        lax.fori_loop(0, npg, body, 0)

    return gather_k


def _mlp_body(flat_ref, x2_ref, W1_ref, b1_ref, W2_ref, b2_ref, Wh_ref,
              Wx_ref, bfx_ref, S_ref, out_ref):
    flat = flat_ref[...]
    h = jnp.dot(flat, W1_ref[...], preferred_element_type=jnp.float32)
    h = jnp.maximum(h + b1_ref[...], 0.0)
    h = jnp.dot(h, W2_ref[...], preferred_element_type=jnp.float32)
    h = jnp.maximum(h + b2_ref[...], 0.0)
    z = (jnp.dot(h, Wh_ref[...], preferred_element_type=jnp.float32)
         + jnp.dot(x2_ref[...], Wx_ref[...], preferred_element_type=jnp.float32)
         + bfx_ref[...])
    z = jnp.maximum(z, 0.0)
    s = jnp.dot(flat, S_ref[...], preferred_element_type=jnp.float32)
    fm = (jnp.sum(s * s, axis=1, keepdims=True)
          - jnp.sum(flat * flat, axis=1, keepdims=True))
    out_ref[...] = jax.nn.sigmoid(z + 0.5 * fm)


def _mlp_call(flat, x2, W1, b1, W2, b2, Wh, Wx, bfx, S, block_b=512):
    b, in_dim = flat.shape
    grid = (b // block_b,)
    full = lambda shape: pl.BlockSpec(shape, lambda i: (0, 0))
    return pl.pallas_call(
        _mlp_body,
        grid=grid,
        in_specs=[
            pl.BlockSpec((block_b, in_dim), lambda i: (i, 0)),
            pl.BlockSpec((block_b, x2.shape[1]), lambda i: (i, 0)),
            full(W1.shape), full(b1.shape), full(W2.shape), full(b2.shape),
            full(Wh.shape), full(Wx.shape), full(bfx.shape), full(S.shape),
        ],
        out_specs=pl.BlockSpec((block_b, 1), lambda i: (i, 0)),
        out_shape=jax.ShapeDtypeStruct((b, 1), jnp.float32),
    )(flat, x2, W1, b1, W2, b2, Wh, Wx, bfx, S)


def kernel(x1, x2, table, W1, b1, W2, b2, Wfx, bfx):
    b = x1.shape[0]
    rows = _make_gather(b)(x1.astype(jnp.int32), table)
    flat = rows.reshape(b, NUM_F * EMB)

    S = jnp.tile(jnp.eye(EMB, dtype=jnp.float32), (NUM_F, 1))
    out = _mlp_call(
        flat, x2, W1, b1.reshape(1, -1), W2, b2.reshape(1, -1),
        Wfx[:W2.shape[1]], Wfx[W2.shape[1]:], bfx.reshape(1, 1), S)
    return out.reshape(-1)
